# baseline (speedup 1.0000x reference)
import functools

import jax
import jax.numpy as jnp
from jax import lax
from jax.experimental import pallas as pl
from jax.experimental.pallas import tpu as pltpu
from jax.experimental.pallas import tpu_sc as plsc

B = 16384
D = 64
K = 20
NC = 2
NS = 16
NW = NC * NS
L = 16
PER_W = B // NW
C = 32
NCHUNK = PER_W // C
NEG_C = C * K
NEG_SUB = NEG_C // 128


def _log_sigmoid(x):
    u = jnp.exp(-jnp.abs(x))
    z = u / (u + 2.0)
    z2 = z * z
    poly = 1.0 + z2 * (1.0 / 3.0 + z2 * (1.0 / 5.0 + z2 * (1.0 / 7.0 + z2 * (1.0 / 9.0))))
    return jnp.minimum(x, 0.0) - 2.0 * z * poly


def _sc_body(in_idx_hbm, out_idx_hbm, neg_idx_hbm, in_tab_hbm, out_tab_hbm,
             out_hbm, idx_in_v, idx_out_v, idx_neg_v, in_rows, out_rows,
             neg_rows, partial_v, sem):
    wid = lax.axis_index("s") * NC + lax.axis_index("c")
    base = wid * PER_W

    pltpu.sync_copy(in_idx_hbm.at[pl.ds(base, PER_W)], idx_in_v)
    pltpu.sync_copy(out_idx_hbm.at[pl.ds(base, PER_W)], idx_out_v)
    pltpu.sync_copy(neg_idx_hbm.at[pl.ds(base * K, PER_W * K)], idx_neg_v)

    lane = lax.iota(jnp.int32, L)

    def chunk(c, total):
        ebase = c * C
        d1 = pltpu.async_copy(
            in_tab_hbm.at[idx_in_v.at[pl.ds(ebase, C)]], in_rows, sem)
        d2 = pltpu.async_copy(
            out_tab_hbm.at[idx_out_v.at[pl.ds(ebase, C)]], out_rows, sem)
        dn = []
        for j in range(NEG_SUB):
            dn.append(pltpu.async_copy(
                out_tab_hbm.at[idx_neg_v.at[pl.ds(ebase * K + j * 128, 128)]],
                neg_rows.at[pl.ds(j * 128, 128)], sem))
        d1.wait()
        d2.wait()
        for d in dn:
            d.wait()

        for g in range(C // L):
            rowv = lane + g * L
            rownk = [rowv * K + k for k in range(K)]
            zeros = jnp.zeros((L,), jnp.float32)

            def jstep(j, accs):
                jv = jnp.full((L,), j, jnp.int32)
                in_j = plsc.load_gather(in_rows, [rowv, jv])
                out_j = plsc.load_gather(out_rows, [rowv, jv])
                new = [accs[0] + in_j * out_j]
                for k in range(K):
                    nk = plsc.load_gather(neg_rows, [rownk[k], jv])
                    new.append(accs[1 + k] + in_j * nk)
                return tuple(new)

            accs = lax.fori_loop(0, D, jstep, (jnp.zeros((L,), jnp.float32),) * (K + 1))
            total = total + _log_sigmoid(accs[0])
            for k in range(K):
                total = total + _log_sigmoid(-accs[1 + k])
        return total

    total = lax.fori_loop(0, NCHUNK, chunk, jnp.zeros((L,), jnp.float32))
    partial_v[...] = total * (1.0 / B)
    pltpu.sync_copy(partial_v, out_hbm.at[pl.ds(wid * L, L)])


@jax.jit
def _sc_call(in_idx, out_idx, neg_flat, in_tab, out_tab):
    mesh = plsc.VectorSubcoreMesh(core_axis_name="c", subcore_axis_name="s")
    f = pl.kernel(
        _sc_body,
        out_type=jax.ShapeDtypeStruct((NW * L,), jnp.float32),
        mesh=mesh,
        scratch_types=[
            pltpu.VMEM((PER_W,), jnp.int32),
            pltpu.VMEM((PER_W,), jnp.int32),
            pltpu.VMEM((PER_W * K,), jnp.int32),
            pltpu.VMEM((C, D), jnp.float32),
            pltpu.VMEM((C, D), jnp.float32),
            pltpu.VMEM((NEG_C, D), jnp.float32),
            pltpu.VMEM((L,), jnp.float32),
            pltpu.SemaphoreType.DMA,
        ],
        compiler_params=pltpu.CompilerParams(
            needs_layout_passes=False, use_tc_tiling_on_sc=False),
    )
    return f(in_idx, out_idx, neg_flat, in_tab, out_tab)


def kernel(input_idx, output_idx, neg_idx, input_vectors, output_vectors):
    partials = _sc_call(
        input_idx.astype(jnp.int32),
        output_idx.astype(jnp.int32),
        neg_idx.astype(jnp.int32).reshape(-1),
        input_vectors,
        output_vectors,
    )
    return jnp.sum(partials)


# speedup vs baseline: 1.1169x; 1.1169x over previous
import jax
import jax.numpy as jnp
from jax import lax
from jax.experimental import pallas as pl
from jax.experimental.pallas import tpu as pltpu
from jax.experimental.pallas import tpu_sc as plsc

B = 16384
D = 64
K = 20
NC = 2
NS = 16
NW = NC * NS
L = 16
PER_W = B // NW
C = 16
NCHUNK = PER_W // C
NEG_C = C * K
CB_W = C + C + NEG_C
TP = 17
NT = K + 1


def _log_sigmoid(x):
    u = jnp.exp(-jnp.abs(x))
    z = u / (u + 2.0)
    z2 = z * z
    poly = 1.0 + z2 * (1.0 / 3.0 + z2 * (1.0 / 5.0 + z2 * (1.0 / 7.0 + z2 * (1.0 / 9.0))))
    return jnp.minimum(x, 0.0) - 2.0 * z * poly


def _sc_body(pin_hbm, pout_hbm, pneg_hbm, cb_hbm, in_tab, out_tab,
             out_hbm, pin_v, pout_v, pneg_v, cb_v, inb, outb, negb, tbuf,
             partial_v, sem):
    wid = lax.axis_index("s") * NC + lax.axis_index("c")
    base = wid * PER_W

    pltpu.sync_copy(pin_hbm.at[pl.ds(base, PER_W)], pin_v)
    pltpu.sync_copy(pout_hbm.at[pl.ds(base, PER_W)], pout_v)
    pltpu.sync_copy(pneg_hbm.at[pl.ds(base * K, PER_W * K)], pneg_v)
    pltpu.sync_copy(cb_hbm.at[pl.ds(wid * NCHUNK, NCHUNK)], cb_v)

    lane = lax.iota(jnp.int32, L)
    zeros = jnp.zeros((L,), jnp.float32)
    for i in range(2 * L * TP // L):
        tbuf[pl.ds(i * L, L)] = zeros

    sign1 = jnp.where(lane == 0, 1.0, -1.0)
    mask2 = jnp.where(lane < NT - L, 1.0, 0.0)

    laneq = [lane + q * L for q in range(D // L)]

    def chunk(c, total):
        ebase = c * C
        d1 = pltpu.async_copy(in_tab.at[pin_v.at[pl.ds(ebase, C)]], inb, sem)
        d2 = pltpu.async_copy(out_tab.at[pout_v.at[pl.ds(ebase, C)]], outb, sem)
        nb = c * NEG_C
        d3 = pltpu.async_copy(out_tab.at[pneg_v.at[pl.ds(nb, 128)]],
                              negb.at[pl.ds(0, 128)], sem)
        d4 = pltpu.async_copy(out_tab.at[pneg_v.at[pl.ds(nb + 128, 128)]],
                              negb.at[pl.ds(128, 128)], sem)
        d5 = pltpu.async_copy(out_tab.at[pneg_v.at[pl.ds(nb + 256, 64)]],
                              negb.at[pl.ds(256, 64)], sem)
        d1.wait()
        d2.wait()
        d3.wait()
        d4.wait()
        d5.wait()

        def elem(e, tot):
            hin = plsc.load_gather(cb_v, [jnp.full((L,), c, jnp.int32),
                                          jnp.full((L,), e, jnp.int32)])
            erow = jnp.full((L,), e, jnp.int32)
            in_q = [plsc.load_gather(inb, [erow, hin + laneq[q]])
                    for q in range(D // L)]
            for t in range(NT):
                if t == 0:
                    hpos = e + C
                    row = outb
                    rowv = erow
                else:
                    hpos = 2 * C + e * K + (t - 1)
                    row = negb
                    rowv = jnp.full((L,), e * K + (t - 1), jnp.int32)
                h = plsc.load_gather(cb_v, [jnp.full((L,), c, jnp.int32),
                                            jnp.full((L,), hpos, jnp.int32)])
                p = in_q[0] * plsc.load_gather(row, [rowv, h + laneq[0]])
                for q in range(1, D // L):
                    p = p + in_q[q] * plsc.load_gather(row, [rowv, h + laneq[q]])
                plsc.store_scatter(tbuf, [lane + t * TP], p)
            r1 = plsc.load_gather(tbuf, [lane * TP])
            r2 = plsc.load_gather(tbuf, [lane * TP + L * TP])
            for j in range(1, L):
                r1 = r1 + plsc.load_gather(tbuf, [lane * TP + j])
                r2 = r2 + plsc.load_gather(tbuf, [lane * TP + L * TP + j])
            return tot + _log_sigmoid(r1 * sign1) + _log_sigmoid(-r2) * mask2

        return lax.fori_loop(0, C, elem, total)

    total = lax.fori_loop(0, NCHUNK, chunk, zeros)
    partial_v[...] = total * (1.0 / B)
    pltpu.sync_copy(partial_v, out_hbm.at[pl.ds(wid * L, L)])


@jax.jit
def _sc_call(pin, pout, pneg, cb, in_tab2, out_tab2):
    mesh = plsc.VectorSubcoreMesh(core_axis_name="c", subcore_axis_name="s")
    f = pl.kernel(
        _sc_body,
        out_type=jax.ShapeDtypeStruct((NW * L,), jnp.float32),
        mesh=mesh,
        scratch_types=[
            pltpu.VMEM((PER_W,), jnp.int32),
            pltpu.VMEM((PER_W,), jnp.int32),
            pltpu.VMEM((PER_W * K,), jnp.int32),
            pltpu.VMEM((NCHUNK, CB_W), jnp.int32),
            pltpu.VMEM((C, 2 * D), jnp.float32),
            pltpu.VMEM((C, 2 * D), jnp.float32),
            pltpu.VMEM((NEG_C, 2 * D), jnp.float32),
            pltpu.VMEM((2 * L * TP,), jnp.float32),
            pltpu.VMEM((L,), jnp.float32),
            pltpu.SemaphoreType.DMA,
        ],
        compiler_params=pltpu.CompilerParams(
            needs_layout_passes=False, use_tc_tiling_on_sc=True),
    )
    return f(pin, pout, pneg, cb, in_tab2, out_tab2)


def kernel(input_idx, output_idx, neg_idx, input_vectors, output_vectors):
    ii = input_idx.astype(jnp.int32)
    oi = output_idx.astype(jnp.int32)
    ni = neg_idx.astype(jnp.int32)
    pin = ii >> 1
    pout = oi >> 1
    pneg = (ni >> 1).reshape(-1)
    h_in = ((ii & 1) * D).reshape(NW * NCHUNK, C)
    h_out = ((oi & 1) * D).reshape(NW * NCHUNK, C)
    h_neg = ((ni & 1) * D).reshape(NW * NCHUNK, NEG_C)
    cb = jnp.concatenate([h_in, h_out, h_neg], axis=1)
    partials = _sc_call(
        pin, pout, pneg, cb,
        input_vectors.reshape(-1, 2 * D),
        output_vectors.reshape(-1, 2 * D),
    )
    return jnp.sum(partials)


# speedup vs baseline: 1.2763x; 1.1427x over previous
import jax
import jax.numpy as jnp
from jax import lax
from jax.experimental import pallas as pl
from jax.experimental.pallas import tpu as pltpu
from jax.experimental.pallas import tpu_sc as plsc

B = 16384
D = 64
K = 20
NC = 2
NS = 16
NW = NC * NS
L = 16
Q = D // L
PER_W = B // NW
C = 32
NCHUNK = PER_W // C
NEG_C = C * K
NEG_SUB = NEG_C // 128
TP = 17
NT = K + 1


def _log_sigmoid(x):
    u = jnp.exp(-jnp.abs(x))
    z = u / (u + 2.0)
    z2 = z * z
    poly = 1.0 + z2 * (1.0 / 3.0 + z2 * (1.0 / 5.0 + z2 * (1.0 / 7.0 + z2 * (1.0 / 9.0))))
    return jnp.minimum(x, 0.0) - 2.0 * z * poly


def _sc_body(in_idx_hbm, out_idx_hbm, neg_idx_hbm, in_tab, out_tab,
             out_hbm, pin_v, pout_v, pneg_v,
             inb0, outb0, negb0, inb1, outb1, negb1,
             tbuf, partial_v, sem0, sem1):
    wid = lax.axis_index("s") * NC + lax.axis_index("c")
    base = wid * PER_W

    pltpu.sync_copy(in_idx_hbm.at[pl.ds(base, PER_W)], pin_v)
    pltpu.sync_copy(out_idx_hbm.at[pl.ds(base, PER_W)], pout_v)
    pltpu.sync_copy(neg_idx_hbm.at[pl.ds(base * K, PER_W * K)], pneg_v)

    lane = lax.iota(jnp.int32, L)
    zeros = jnp.zeros((L,), jnp.float32)
    laneq = [lane + q * L for q in range(Q)]
    for i in range(2 * L * TP // L):
        tbuf[pl.ds(i * L, L)] = zeros

    sign1 = jnp.where(lane == 0, 1.0, -1.0)
    mask2 = jnp.where(lane < NT - L, 1.0, 0.0)

    def issue(c, inb, outb, negb, sem):
        ebase = c * C
        pltpu.async_copy(in_tab.at[pin_v.at[pl.ds(ebase, C)]], inb, sem)
        pltpu.async_copy(out_tab.at[pout_v.at[pl.ds(ebase, C)]], outb, sem)
        nb = c * NEG_C
        for j in range(NEG_SUB):
            pltpu.async_copy(out_tab.at[pneg_v.at[pl.ds(nb + j * 128, 128)]],
                             negb.at[pl.ds(j * 128, 128)], sem)

    def drain(inb, outb, negb, sem):
        pltpu.make_async_copy(in_tab.at[pin_v.at[pl.ds(0, C)]], inb, sem).wait()
        pltpu.make_async_copy(out_tab.at[pout_v.at[pl.ds(0, C)]], outb, sem).wait()
        for j in range(NEG_SUB):
            pltpu.make_async_copy(
                out_tab.at[pneg_v.at[pl.ds(j * 128, 128)]],
                negb.at[pl.ds(j * 128, 128)], sem).wait()

    def compute(inb, outb, negb, total):
        def elem(e, tot):
            erow = jnp.full((L,), e, jnp.int32)
            in_q = [plsc.load_gather(inb, [erow, laneq[q]]) for q in range(Q)]
            for t in range(NT):
                if t == 0:
                    row = outb
                    rowv = erow
                else:
                    row = negb
                    rowv = jnp.full((L,), e * K + (t - 1), jnp.int32)
                p = in_q[0] * plsc.load_gather(row, [rowv, laneq[0]])
                for q in range(1, Q):
                    p = p + in_q[q] * plsc.load_gather(row, [rowv, laneq[q]])
                plsc.store_scatter(tbuf, [lane + t * TP], p)
            r1 = plsc.load_gather(tbuf, [lane * TP])
            r2 = plsc.load_gather(tbuf, [lane * TP + L * TP])
            for j in range(1, L):
                r1 = r1 + plsc.load_gather(tbuf, [lane * TP + j])
                r2 = r2 + plsc.load_gather(tbuf, [lane * TP + L * TP + j])
            return tot + _log_sigmoid(r1 * sign1) + _log_sigmoid(-r2) * mask2

        return lax.fori_loop(0, C, elem, total)

    issue(0, inb0, outb0, negb0, sem0)

    def pair(i, total):
        c0 = 2 * i
        issue(c0 + 1, inb1, outb1, negb1, sem1)
        drain(inb0, outb0, negb0, sem0)
        total = compute(inb0, outb0, negb0, total)
        cn = lax.select(c0 + 2 < NCHUNK, c0 + 2, 0)
        issue(cn, inb0, outb0, negb0, sem0)
        drain(inb1, outb1, negb1, sem1)
        return compute(inb1, outb1, negb1, total)

    total = lax.fori_loop(0, NCHUNK // 2, pair, zeros)
    drain(inb0, outb0, negb0, sem0)

    partial_v[...] = total * (1.0 / B)
    pltpu.sync_copy(partial_v, out_hbm.at[pl.ds(wid * L, L)])


@jax.jit
def _sc_call(in_idx, out_idx, neg_flat, in_tab, out_tab):
    mesh = plsc.VectorSubcoreMesh(core_axis_name="c", subcore_axis_name="s")
    f = pl.kernel(
        _sc_body,
        out_type=jax.ShapeDtypeStruct((NW * L,), jnp.float32),
        mesh=mesh,
        scratch_types=[
            pltpu.VMEM((PER_W,), jnp.int32),
            pltpu.VMEM((PER_W,), jnp.int32),
            pltpu.VMEM((PER_W * K,), jnp.int32),
            pltpu.VMEM((C, D), jnp.float32),
            pltpu.VMEM((C, D), jnp.float32),
            pltpu.VMEM((NEG_C, D), jnp.float32),
            pltpu.VMEM((C, D), jnp.float32),
            pltpu.VMEM((C, D), jnp.float32),
            pltpu.VMEM((NEG_C, D), jnp.float32),
            pltpu.VMEM((2 * L * TP,), jnp.float32),
            pltpu.VMEM((L,), jnp.float32),
            pltpu.SemaphoreType.DMA,
            pltpu.SemaphoreType.DMA,
        ],
        compiler_params=pltpu.CompilerParams(
            needs_layout_passes=False, use_tc_tiling_on_sc=False),
    )
    return f(in_idx, out_idx, neg_flat, in_tab, out_tab)


def kernel(input_idx, output_idx, neg_idx, input_vectors, output_vectors):
    partials = _sc_call(
        input_idx.astype(jnp.int32),
        output_idx.astype(jnp.int32),
        neg_idx.astype(jnp.int32).reshape(-1),
        input_vectors,
        output_vectors,
    )
    return jnp.sum(partials)


# speedup vs baseline: 1.5012x; 1.1762x over previous
import jax
import jax.numpy as jnp
from jax import lax
from jax.experimental import pallas as pl
from jax.experimental.pallas import tpu as pltpu
from jax.experimental.pallas import tpu_sc as plsc

B = 16384
D = 64
W = 2 * D
K = 20
NC = 2
NS = 16
NW = NC * NS
L = 16
Q = D // L
PER_W = B // NW
C = 16
NCHUNK = PER_W // C
IO_C = 2 * C
NEG_C = C * K
TP = 17
NT = K + 1


def _log_sigmoid(x):
    u = jnp.exp(-jnp.abs(x))
    z = u / (u + 2.0)
    z2 = z * z
    poly = 1.0 + z2 * (1.0 / 3.0 + z2 * (1.0 / 5.0 + z2 * (1.0 / 7.0 + z2 * (1.0 / 9.0))))
    return jnp.minimum(x, 0.0) - 2.0 * z * poly


def _sc_body(io_idx_hbm, neg_idx_hbm, tab,
             out_hbm, pio_v, pneg_v,
             iob0, negb0, iob1, negb1,
             tbuf, partial_v, sem0, sem1):
    wid = lax.axis_index("s") * NC + lax.axis_index("c")
    base = wid * PER_W

    pltpu.sync_copy(io_idx_hbm.at[pl.ds(base * 2, PER_W * 2)], pio_v)
    pltpu.sync_copy(neg_idx_hbm.at[pl.ds(base * K, PER_W * K)], pneg_v)

    lane = lax.iota(jnp.int32, L)
    zeros = jnp.zeros((L,), jnp.float32)
    laneq_in = [lane + q * L for q in range(Q)]
    laneq_out = [lane + D + q * L for q in range(Q)]
    for i in range(2 * L * TP // L):
        tbuf[pl.ds(i * L, L)] = zeros

    sign1 = jnp.where(lane == 0, 1.0, -1.0)
    mask2 = jnp.where(lane < NT - L, 1.0, 0.0)

    def issue(c, iob, negb, sem):
        pltpu.async_copy(tab.at[pio_v.at[pl.ds(c * IO_C, IO_C)]], iob, sem)
        nb = c * NEG_C
        pltpu.async_copy(tab.at[pneg_v.at[pl.ds(nb, 128)]],
                         negb.at[pl.ds(0, 128)], sem)
        pltpu.async_copy(tab.at[pneg_v.at[pl.ds(nb + 128, 128)]],
                         negb.at[pl.ds(128, 128)], sem)
        pltpu.async_copy(tab.at[pneg_v.at[pl.ds(nb + 256, 64)]],
                         negb.at[pl.ds(256, 64)], sem)

    def drain(iob, negb, sem):
        pltpu.make_async_copy(tab.at[pio_v.at[pl.ds(0, IO_C)]], iob, sem).wait()
        pltpu.make_async_copy(tab.at[pneg_v.at[pl.ds(0, 128)]],
                              negb.at[pl.ds(0, 128)], sem).wait()
        pltpu.make_async_copy(tab.at[pneg_v.at[pl.ds(128, 128)]],
                              negb.at[pl.ds(128, 128)], sem).wait()
        pltpu.make_async_copy(tab.at[pneg_v.at[pl.ds(256, 64)]],
                              negb.at[pl.ds(256, 64)], sem).wait()

    def compute(iob, negb, total):
        def elem(e, tot):
            erow = jnp.full((L,), e, jnp.int32)
            in_q = [plsc.load_gather(iob, [erow, laneq_in[q]])
                    for q in range(Q)]
            for t in range(NT):
                if t == 0:
                    row = iob
                    rowv = erow + C
                else:
                    row = negb
                    rowv = jnp.full((L,), e * K + (t - 1), jnp.int32)
                p = in_q[0] * plsc.load_gather(row, [rowv, laneq_out[0]])
                for q in range(1, Q):
                    p = p + in_q[q] * plsc.load_gather(row, [rowv, laneq_out[q]])
                plsc.store_scatter(tbuf, [lane + t * TP], p)
            r1 = plsc.load_gather(tbuf, [lane * TP])
            r2 = plsc.load_gather(tbuf, [lane * TP + L * TP])
            for j in range(1, L):
                r1 = r1 + plsc.load_gather(tbuf, [lane * TP + j])
                r2 = r2 + plsc.load_gather(tbuf, [lane * TP + L * TP + j])
            return tot + _log_sigmoid(r1 * sign1) + _log_sigmoid(-r2) * mask2

        return lax.fori_loop(0, C, elem, total)

    issue(0, iob0, negb0, sem0)

    def pair(i, total):
        c0 = 2 * i
        issue(c0 + 1, iob1, negb1, sem1)
        drain(iob0, negb0, sem0)
        total = compute(iob0, negb0, total)
        cn = lax.select(c0 + 2 < NCHUNK, c0 + 2, 0)
        issue(cn, iob0, negb0, sem0)
        drain(iob1, negb1, sem1)
        return compute(iob1, negb1, total)

    total = lax.fori_loop(0, NCHUNK // 2, pair, zeros)
    drain(iob0, negb0, sem0)

    partial_v[...] = total * (1.0 / B)
    pltpu.sync_copy(partial_v, out_hbm.at[pl.ds(wid * L, L)])


@jax.jit
def _sc_call(io_idx, neg_flat, tab):
    mesh = plsc.VectorSubcoreMesh(core_axis_name="c", subcore_axis_name="s")
    f = pl.kernel(
        _sc_body,
        out_type=jax.ShapeDtypeStruct((NW * L,), jnp.float32),
        mesh=mesh,
        scratch_types=[
            pltpu.VMEM((2 * PER_W,), jnp.int32),
            pltpu.VMEM((PER_W * K,), jnp.int32),
            pltpu.VMEM((IO_C, W), jnp.float32),
            pltpu.VMEM((NEG_C, W), jnp.float32),
            pltpu.VMEM((IO_C, W), jnp.float32),
            pltpu.VMEM((NEG_C, W), jnp.float32),
            pltpu.VMEM((2 * L * TP,), jnp.float32),
            pltpu.VMEM((L,), jnp.float32),
            pltpu.SemaphoreType.DMA,
            pltpu.SemaphoreType.DMA,
        ],
        compiler_params=pltpu.CompilerParams(
            needs_layout_passes=False, use_tc_tiling_on_sc=True),
    )
    return f(io_idx, neg_flat, tab)


def kernel(input_idx, output_idx, neg_idx, input_vectors, output_vectors):
    ii = input_idx.astype(jnp.int32)
    oi = output_idx.astype(jnp.int32)
    ni = neg_idx.astype(jnp.int32)
    tab = jnp.concatenate([input_vectors, output_vectors], axis=1)
    io_idx = jnp.concatenate(
        [ii.reshape(-1, C), oi.reshape(-1, C)], axis=1).reshape(-1)
    partials = _sc_call(io_idx, ni.reshape(-1), tab)
    return jnp.sum(partials)
